# baseline (speedup 1.0000x reference)
import functools

import jax
import jax.numpy as jnp
from jax import lax
from jax.experimental import pallas as pl
from jax.experimental.pallas import tpu as pltpu
from jax.experimental.pallas import tpu_sc as plsc

_B, _T, _I = 8, 8192, 256
_L = 16


_S = 8
_BT = 1024
_TROW = _B * _T // (2 * _S)
_NBLK = _TROW // _BT


def _argmax_body(*refs):
    scores_refs, out_refs = refs[:_S], refs[_S:]
    rows = 512
    iota_f = lax.broadcasted_iota(jnp.int32, (rows, _I), 1).astype(jnp.float32)

    for s in range(_S):
        for i in range(_BT // rows):
            x = scores_refs[s][0, pl.ds(i * rows, rows), :]
            m = jnp.max(x, axis=1, keepdims=True)
            cand = jnp.where(x == m, iota_f, float(_I))
            amin = jnp.min(cand, axis=1, keepdims=True)
            out_refs[s][0, pl.ds(i * rows, rows), :] = amin.astype(jnp.int32)


def _tc_argmax_half(scores16, half, interpret=False):
    def in_spec(s):
        return pl.BlockSpec(
            (1, _BT, _I), lambda b, t, s=s: (half * _S + s, t, 0))

    out_spec = pl.BlockSpec((1, _BT, 1), lambda b, t: (b * _NBLK + t, 0, 0))

    out = pl.pallas_call(
        _argmax_body,
        grid=(1, _NBLK),
        in_specs=[in_spec(s) for s in range(_S)],
        out_specs=[out_spec for _ in range(_S)],
        out_shape=[jax.ShapeDtypeStruct((_NBLK, _BT, 1), jnp.int32)
                   for _ in range(_S)],
        interpret=interpret,
    )(*([scores16] * _S))
    inst = jnp.stack([o.reshape(_TROW) for o in out])
    return inst.reshape(_B // 2, _T)


_NS = 16
_W = 8
_TQ = _T // _W


def _make_sc_body(half):
    def _sc_body(types_hbm, inst_hbm, partials_hbm, total_hbm,
                 types_v, inst_v, pres_v, grp_v, stage_v, red_v, shared):
        c = lax.axis_index("c")
        s = lax.axis_index("s")
        k = s // _W
        q = s % _W
        bl = 2 * k + c
        bg = half * (_B // 2) + bl

        pltpu.sync_copy(types_hbm.at[bg, pl.ds(q * _TQ, _TQ)], types_v)
        pltpu.sync_copy(inst_hbm.at[bl, pl.ds(q * _TQ, _TQ)], inst_v)

        zeros = jnp.zeros((_L,), jnp.int32)
        for i in range(_I // _L):
            pres_v[pl.ds(i * _L, _L)] = zeros
        ones = jnp.ones((_L,), jnp.int32)

        n1 = zeros
        for t in range(_TQ // _L):
            tv = types_v[pl.ds(t * _L, _L)]
            iv = inst_v[pl.ds(t * _L, _L)]
            m1 = tv == 1
            plsc.store_scatter(pres_v, [iv], ones, mask=m1)
            n1 = n1 + jnp.where(m1, 1, 0).astype(jnp.int32)

        pltpu.sync_copy(pres_v, shared.at[s])
        plsc.subcore_barrier()
        pltpu.sync_copy(shared.at[pl.ds(k * _W, _W)], grp_v)
        for i in range(_I // _L):
            acc = jnp.zeros((_L,), jnp.int32)
            for r in range(_W):
                acc = acc + grp_v[r, pl.ds(i * _L, _L)]
            pres_v[pl.ds(i * _L, _L)] = acc

        u1 = zeros
        for i in range(_I // _L):
            u1 = u1 + jnp.where(
                pres_v[pl.ds(i * _L, _L)] > 0, 1, 0).astype(jnp.int32)
        u1 = jnp.where(q == 0, u1, zeros)

        r22 = zeros
        for t in range(_TQ // _L):
            tv = types_v[pl.ds(t * _L, _L)]
            iv = inst_v[pl.ds(t * _L, _L)]
            g = plsc.load_gather(pres_v, [iv])
            miss = jnp.logical_and(tv == 3, g == 0)
            r22 = r22 + jnp.where(miss, 1, 0).astype(jnp.int32)

        stage_v[...] = n1 - u1 + r22
        pltpu.sync_copy(stage_v, partials_hbm.at[pl.ds((c * _NS + s) * _L, _L)])

        plsc.subcore_barrier()

        @pl.when(s == 0)
        def _():
            pltpu.sync_copy(
                partials_hbm.at[pl.ds(c * _NS * _L, _NS * _L)], red_v)
            acc = jnp.zeros((_L,), jnp.int32)
            for i in range(_NS):
                acc = acc + red_v[pl.ds(i * _L, _L)]
            total = jnp.sum(acc)
            stage_v[...] = jnp.full((_L,), total, jnp.int32)
            pltpu.sync_copy(stage_v, total_hbm.at[pl.ds(c * _L, _L)])

    return _sc_body


@functools.cache
def _sc_binning(half):
    return pl.kernel(
        _make_sc_body(half),
        out_type=(
            jax.ShapeDtypeStruct((2 * _NS * _L,), jnp.int32),
            jax.ShapeDtypeStruct((2 * _L,), jnp.int32),
        ),
        mesh=plsc.VectorSubcoreMesh(core_axis_name="c", subcore_axis_name="s"),
        compiler_params=pltpu.CompilerParams(needs_layout_passes=False),
        scratch_types=[
            pltpu.VMEM((_TQ,), jnp.int32),
            pltpu.VMEM((_TQ,), jnp.int32),
            pltpu.VMEM((_I,), jnp.int32),
            pltpu.VMEM((_W, _I), jnp.int32),
            pltpu.VMEM((_L,), jnp.int32),
            pltpu.VMEM((_NS * _L,), jnp.int32),
            pltpu.VMEM_SHARED((_NS, _I), jnp.int32),
        ],
    )


def kernel(max_pred_types, instrument_scores):
    scores16 = instrument_scores.reshape(2 * _S, _TROW, _I)
    inst0 = _tc_argmax_half(scores16, 0)
    _, tot0 = _sc_binning(0)(max_pred_types, inst0)
    inst1 = _tc_argmax_half(scores16, 1)
    _, tot1 = _sc_binning(1)(max_pred_types, inst1)
    return tot0[0] + tot0[_L] + tot1[0] + tot1[_L]


# speedup vs baseline: 1.0634x; 1.0634x over previous
import functools

import jax
import jax.numpy as jnp
from jax import lax
from jax.experimental import pallas as pl
from jax.experimental.pallas import tpu as pltpu
from jax.experimental.pallas import tpu_sc as plsc

_B, _T, _I = 8, 8192, 256
_BT = 2048
_L = 16


_S = 8


def _argmax_body(*refs):
    scores_refs, out_refs = refs[:_S], refs[_S:]
    rows = 512
    iota_f = lax.broadcasted_iota(jnp.int32, (rows, _I), 1).astype(jnp.float32)

    for s in range(_S):
        for i in range(_BT // rows):
            x = scores_refs[s][0, pl.ds(i * rows, rows), :]
            m = jnp.max(x, axis=1, keepdims=True)
            cand = jnp.where(x == m, iota_f, float(_I))
            amin = jnp.min(cand, axis=1, keepdims=True)
            out_refs[s][0, pl.ds(i * rows, rows), :] = amin.astype(jnp.int32)


def _tc_argmax(scores, interpret=False):
    nblk = _T // _BT
    ngrp = _B // _S

    def in_spec(s):
        return pl.BlockSpec((1, _BT, _I), lambda b, t, s=s: (b * _S + s, t, 0))

    out_spec = pl.BlockSpec((1, _BT, 1), lambda b, t: (b * nblk + t, 0, 0))

    out = pl.pallas_call(
        _argmax_body,
        grid=(ngrp, nblk),
        in_specs=[in_spec(s) for s in range(_S)],
        out_specs=[out_spec for _ in range(_S)],
        out_shape=[jax.ShapeDtypeStruct((ngrp * nblk, _BT, 1), jnp.int32)
                   for _ in range(_S)],
        interpret=interpret,
    )(*([scores] * _S))
    inst = jnp.stack([o.reshape(ngrp, _T) for o in out], axis=1)
    return inst.reshape(_B, _T)


_NS = 16
_W = 4
_TQ = _T // _W
_U = 8


def _sc_body(types_hbm, inst_hbm, partials_hbm, total_hbm,
             types_v, inst_v, pres_v, tmp_v, stage_v, red_v, shared):
    c = lax.axis_index("c")
    s = lax.axis_index("s")
    k = s // _W
    q = s % _W
    b = 2 * k + c

    pltpu.sync_copy(types_hbm.at[b, pl.ds(q * _TQ, _TQ)], types_v)
    pltpu.sync_copy(inst_hbm.at[b, pl.ds(q * _TQ, _TQ)], inst_v)

    zeros = jnp.zeros((_L,), jnp.int32)
    for i in range(_I // _L):
        pres_v[pl.ds(i * _L, _L)] = zeros
    ones = jnp.ones((_L,), jnp.int32)

    def pass_scatter(t, n1_acc):
        for u in range(_U):
            tv = types_v[pl.ds((t * _U + u) * _L, _L)]
            iv = inst_v[pl.ds((t * _U + u) * _L, _L)]
            m1 = tv == 1
            plsc.store_scatter(pres_v, [iv], ones, mask=m1)
            n1_acc = n1_acc + jnp.where(m1, 1, 0).astype(jnp.int32)
        return n1_acc

    n1 = lax.fori_loop(0, _TQ // (_L * _U), pass_scatter, zeros)

    pltpu.sync_copy(pres_v, shared.at[s])
    plsc.subcore_barrier()
    for j in range(1, _W):
        pltpu.sync_copy(shared.at[k * _W + ((q + j) % _W)], tmp_v)
        for i in range(_I // _L):
            sl = pl.ds(i * _L, _L)
            pres_v[sl] = pres_v[sl] + tmp_v[sl]

    u1 = jnp.zeros((_L,), jnp.int32)
    for i in range(_I // _L):
        u1 = u1 + jnp.where(pres_v[pl.ds(i * _L, _L)] > 0, 1, 0).astype(jnp.int32)
    u1 = jnp.where(q == 0, u1, zeros)

    def pass_gather(t, r_acc):
        for u in range(_U):
            tv = types_v[pl.ds((t * _U + u) * _L, _L)]
            iv = inst_v[pl.ds((t * _U + u) * _L, _L)]
            g = plsc.load_gather(pres_v, [iv])
            miss = jnp.logical_and(tv == 3, g == 0)
            r_acc = r_acc + jnp.where(miss, 1, 0).astype(jnp.int32)
        return r_acc

    r22 = lax.fori_loop(0, _TQ // (_L * _U), pass_gather, zeros)

    stage_v[...] = n1 - u1 + r22
    pltpu.sync_copy(stage_v, partials_hbm.at[pl.ds((c * _NS + s) * _L, _L)])

    plsc.subcore_barrier()

    @pl.when(s == 0)
    def _():
        pltpu.sync_copy(partials_hbm.at[pl.ds(c * _NS * _L, _NS * _L)], red_v)
        acc = jnp.zeros((_L,), jnp.int32)
        for i in range(_NS):
            acc = acc + red_v[pl.ds(i * _L, _L)]
        total = jnp.sum(acc)
        stage_v[...] = jnp.full((_L,), total, jnp.int32)
        pltpu.sync_copy(stage_v, total_hbm.at[pl.ds(c * _L, _L)])


@functools.cache
def _sc_binning():
    return pl.kernel(
        _sc_body,
        out_type=(
            jax.ShapeDtypeStruct((2 * _NS * _L,), jnp.int32),
            jax.ShapeDtypeStruct((2 * _L,), jnp.int32),
        ),
        mesh=plsc.VectorSubcoreMesh(core_axis_name="c", subcore_axis_name="s"),
        compiler_params=pltpu.CompilerParams(needs_layout_passes=False),
        scratch_types=[
            pltpu.VMEM((_TQ,), jnp.int32),
            pltpu.VMEM((_TQ,), jnp.int32),
            pltpu.VMEM((_I,), jnp.int32),
            pltpu.VMEM((_I,), jnp.int32),
            pltpu.VMEM((_L,), jnp.int32),
            pltpu.VMEM((_NS * _L,), jnp.int32),
            pltpu.VMEM_SHARED((_NS, _I), jnp.int32),
        ],
    )


def kernel(max_pred_types, instrument_scores):
    inst = _tc_argmax(instrument_scores)
    _, total_vec = _sc_binning()(max_pred_types, inst)
    return total_vec[0] + total_vec[_L]
